# Initial kernel scaffold; baseline (speedup 1.0000x reference)
import jax
import jax.numpy as jnp
from jax.experimental import pallas as pl


def kernel(x, edge_index, batch, W1, b1, W2, b2, Wf1, bf1, Wf2, bf2):
    raise NotImplementedError("write your pallas kernel here")


# speedup vs baseline: 26.3867x; 26.3867x over previous
import functools

import jax
import jax.numpy as jnp
from jax import lax
from jax.experimental import pallas as pl
from jax.experimental.pallas import tpu as pltpu
from jax.experimental.pallas import tpu_sc as plsc

N = 10000
E = 320000
D = 128
H = 64
G = 64
OUT = 2

C = 100
NCH = E // C
NW = 32
CHW = NCH // NW
NP = 10240
RPS = NP // 16
DEGW = 16

_mesh = plsc.VectorSubcoreMesh(core_axis_name="c", subcore_axis_name="s")


@functools.partial(
    pl.kernel,
    out_type=jax.ShapeDtypeStruct((2, NP, DEGW), jnp.float32),
    mesh=_mesh,
    scratch_types=[
        pltpu.VMEM((CHW, C), jnp.int32),
        pltpu.VMEM((C, DEGW), jnp.float32),
        pltpu.VMEM_SHARED((NP, DEGW), jnp.float32),
    ],
    compiler_params=pltpu.CompilerParams(use_tc_tiling_on_sc=False),
)
def _sc_degree(dst_hbm, ones_hbm, zeros_hbm, out_hbm, dst_v, ones_v, acc):
    cid = lax.axis_index("c")
    sid = lax.axis_index("s")
    wid = sid * 2 + cid
    pltpu.sync_copy(zeros_hbm.at[pl.ds(sid * RPS, RPS)],
                    acc.at[pl.ds(sid * RPS, RPS)])
    pltpu.sync_copy(dst_hbm.at[wid], dst_v)
    pltpu.sync_copy(ones_hbm, ones_v)
    plsc.subcore_barrier()

    def body(j, carry):
        pltpu.sync_copy(ones_v, acc.at[dst_v.at[j]], add=True)
        return carry

    lax.fori_loop(0, CHW, body, 0)
    plsc.subcore_barrier()
    pltpu.sync_copy(acc.at[pl.ds(sid * RPS, RPS)],
                    out_hbm.at[cid, pl.ds(sid * RPS, RPS)])


@functools.partial(
    pl.kernel,
    out_type=jax.ShapeDtypeStruct((2, NP, H), jnp.float32),
    mesh=_mesh,
    scratch_types=[
        pltpu.VMEM((CHW, C), jnp.int32),
        pltpu.VMEM((CHW, C), jnp.int32),
        pltpu.VMEM((C, H), jnp.float32),
        pltpu.VMEM_SHARED((NP, H), jnp.float32),
        pltpu.SemaphoreType.DMA,
    ],
    compiler_params=pltpu.CompilerParams(use_tc_tiling_on_sc=False),
)
def _sc_scatter(h_hbm, src_hbm, dst_hbm, zeros_hbm, out_hbm,
                src_v, dst_v, rows_v, acc, sem):
    cid = lax.axis_index("c")
    sid = lax.axis_index("s")
    wid = sid * 2 + cid
    pltpu.sync_copy(zeros_hbm.at[pl.ds(sid * RPS, RPS)],
                    acc.at[pl.ds(sid * RPS, RPS)])
    pltpu.sync_copy(src_hbm.at[wid], src_v)
    pltpu.sync_copy(dst_hbm.at[wid], dst_v)
    plsc.subcore_barrier()

    def body(j, carry):
        pltpu.async_copy(h_hbm.at[src_v.at[j]], rows_v, sem).wait()
        pltpu.sync_copy(rows_v, acc.at[dst_v.at[j]], add=True)
        return carry

    lax.fori_loop(0, CHW, body, 0)
    plsc.subcore_barrier()
    pltpu.sync_copy(acc.at[pl.ds(sid * RPS, RPS)],
                    out_hbm.at[cid, pl.ds(sid * RPS, RPS)])


def _tc_a_body(x_ref, w1_ref, degp_ref, h_ref, dinv_ref):
    deg = (degp_ref[0] + degp_ref[1])[:N] + 1.0
    dinv = lax.rsqrt(deg)
    dinv_ref[...] = dinv
    h = jnp.dot(x_ref[...], w1_ref[...], preferred_element_type=jnp.float32)
    h_ref[...] = h * dinv[:, :1]


def _tc_b_body(raw_ref, h1_ref, dinv_ref, b1_ref, w2_ref, h2_ref):
    dinv = dinv_ref[:, :1]
    raw = (raw_ref[0] + raw_ref[1])[:N]
    z1 = jnp.maximum((raw + h1_ref[...]) * dinv + b1_ref[...], 0.0)
    h2 = jnp.dot(z1, w2_ref[...], preferred_element_type=jnp.float32)
    h2_ref[...] = h2 * dinv


def _tc_c_body(raw_ref, h2_ref, dinv_ref, b2_ref, batch_ref,
               wf1_ref, bf1_ref, wf2_ref, bf2_ref, out_ref):
    dinv = dinv_ref[:, :1]
    raw = (raw_ref[0] + raw_ref[1])[:N]
    z2 = jnp.maximum((raw + h2_ref[...]) * dinv + b2_ref[...], 0.0)
    gids = lax.broadcasted_iota(jnp.int32, (G, N), 0)
    onehot = (gids == batch_ref[...]).astype(jnp.float32)
    sums = jnp.dot(onehot, z2, preferred_element_type=jnp.float32)
    cnts = jnp.sum(onehot, axis=1, keepdims=True)
    pooled = sums / jnp.maximum(cnts, 1.0)
    hfc = jnp.maximum(
        jnp.dot(pooled, wf1_ref[...], preferred_element_type=jnp.float32)
        + bf1_ref[...], 0.0)
    out_ref[...] = (jnp.dot(hfc, wf2_ref[...],
                            preferred_element_type=jnp.float32)
                    + bf2_ref[...])


_tc_a = pl.pallas_call(
    _tc_a_body,
    out_shape=(jax.ShapeDtypeStruct((N, H), jnp.float32),
               jax.ShapeDtypeStruct((N, DEGW), jnp.float32)),
)

_tc_b = pl.pallas_call(
    _tc_b_body,
    out_shape=jax.ShapeDtypeStruct((N, H), jnp.float32),
)

_tc_c = pl.pallas_call(
    _tc_c_body,
    out_shape=jax.ShapeDtypeStruct((G, OUT), jnp.float32),
)


def kernel(x, edge_index, batch, W1, b1, W2, b2, Wf1, bf1, Wf2, bf2):
    src = edge_index[0].reshape(NW, CHW, C)
    dst = edge_index[1].reshape(NW, CHW, C)
    zeros_h = jnp.zeros((NP, H), jnp.float32)
    zeros_d = jnp.zeros((NP, DEGW), jnp.float32)
    ones_d = jnp.ones((C, DEGW), jnp.float32)

    degp = _sc_degree(dst, ones_d, zeros_d)
    h1s, dinv = _tc_a(x, W1, degp)
    raw1 = _sc_scatter(h1s, src, dst, zeros_h)
    h2s = _tc_b(raw1, h1s, dinv, b1.reshape(1, H), W2)
    raw2 = _sc_scatter(h2s, src, dst, zeros_h)
    out = _tc_c(raw2, h2s, dinv, b2.reshape(1, H), batch.reshape(1, N),
                Wf1, bf1.reshape(1, H // 2), Wf2, bf2.reshape(1, OUT))
    return out


# speedup vs baseline: 43.4131x; 1.6453x over previous
import functools

import jax
import jax.numpy as jnp
from jax import lax
from jax.experimental import pallas as pl
from jax.experimental.pallas import tpu as pltpu
from jax.experimental.pallas import tpu_sc as plsc

N = 10000
E = 320000
D = 128
H = 64
G = 64
OUT = 2

C = 125
NW = 32
CHW = E // (NW * C)
NP = 10240
RPS = NP // 16
DEGW = 16
NSLOT = 8

_mesh = plsc.VectorSubcoreMesh(core_axis_name="c", subcore_axis_name="s")
_sc_params = pltpu.CompilerParams(use_tc_tiling_on_sc=False)


@functools.partial(
    pl.kernel,
    out_type=jax.ShapeDtypeStruct((2, NP, DEGW), jnp.float32),
    mesh=_mesh,
    scratch_types=[
        pltpu.VMEM((CHW, C), jnp.int32),
        pltpu.VMEM((C, DEGW), jnp.float32),
        pltpu.VMEM_SHARED((NP, DEGW), jnp.float32),
        [pltpu.SemaphoreType.DMA] * 4,
    ],
    compiler_params=_sc_params,
)
def _sc_degree(dst_hbm, ones_hbm, zeros_hbm, out_hbm, dst_v, ones_v, acc, sems):
    cid = lax.axis_index("c")
    sid = lax.axis_index("s")
    wid = sid * 2 + cid
    pltpu.sync_copy(zeros_hbm.at[pl.ds(sid * RPS, RPS)],
                    acc.at[pl.ds(sid * RPS, RPS)])
    pltpu.sync_copy(dst_hbm.at[wid], dst_v)
    pltpu.sync_copy(ones_hbm, ones_v)
    plsc.subcore_barrier()

    def _scat(j, b):
        pltpu.async_copy(ones_v, acc.at[dst_v.at[j]], sems[b], add=True)

    def _drain(j, b):
        pltpu.make_async_copy(ones_v, acc.at[dst_v.at[j]], sems[b]).wait()

    for b in range(4):
        _scat(b, b)

    def body(g, carry):
        for b in range(4):
            j = 4 + g * 4 + b
            _drain(j, b)
            _scat(j, b)
        return carry

    lax.fori_loop(0, (CHW - 4) // 4, body, 0)
    for b in range(4):
        _drain(0, b)
    plsc.subcore_barrier()
    pltpu.sync_copy(acc.at[pl.ds(sid * RPS, RPS)],
                    out_hbm.at[cid, pl.ds(sid * RPS, RPS)])


@functools.partial(
    pl.kernel,
    out_type=jax.ShapeDtypeStruct((2, NP, H), jnp.float32),
    mesh=_mesh,
    scratch_types=[
        pltpu.VMEM((CHW, C), jnp.int32),
        pltpu.VMEM((CHW, C), jnp.int32),
        pltpu.VMEM((NSLOT, C, H), jnp.float32),
        pltpu.VMEM_SHARED((NP, H), jnp.float32),
        [pltpu.SemaphoreType.DMA] * NSLOT,
        [pltpu.SemaphoreType.DMA] * NSLOT,
    ],
    compiler_params=_sc_params,
)
def _sc_scatter(h_hbm, src_hbm, dst_hbm, zeros_hbm, out_hbm,
                src_v, dst_v, rows_v, acc, gsems, ssems):
    cid = lax.axis_index("c")
    sid = lax.axis_index("s")
    wid = sid * 2 + cid
    pltpu.sync_copy(zeros_hbm.at[pl.ds(sid * RPS, RPS)],
                    acc.at[pl.ds(sid * RPS, RPS)])
    pltpu.sync_copy(src_hbm.at[wid], src_v)
    pltpu.sync_copy(dst_hbm.at[wid], dst_v)
    plsc.subcore_barrier()

    def _gather(j, b):
        pltpu.async_copy(h_hbm.at[src_v.at[j]], rows_v.at[b], gsems[b])

    def _gwait(j, b):
        pltpu.make_async_copy(h_hbm.at[src_v.at[j]], rows_v.at[b],
                              gsems[b]).wait()

    def _scat(j, b):
        pltpu.async_copy(rows_v.at[b], acc.at[dst_v.at[j]], ssems[b], add=True)

    def _swait(j, b):
        pltpu.make_async_copy(rows_v.at[b], acc.at[dst_v.at[j]],
                              ssems[b]).wait()

    for b in range(4):
        _gather(b, b)
    for j in range(4):
        _gwait(j, j)
        _scat(j, j)
        _gather(j + 4, j + 4)

    def body(g, carry):
        for boff in range(8):
            j = 4 + g * 8 + boff
            b = (4 + boff) % NSLOT
            b4 = boff % NSLOT
            _gwait(j, b)
            _scat(j, b)
            _swait(j - 4, b4)
            _gather(j + 4, b4)
        return carry

    lax.fori_loop(0, (CHW - 8) // 8, body, 0)
    for j in range(CHW - 4, CHW):
        b = j % NSLOT
        _gwait(j, b)
        _scat(j, b)
    for b in range(NSLOT):
        _swait(0, b)
    plsc.subcore_barrier()
    pltpu.sync_copy(acc.at[pl.ds(sid * RPS, RPS)],
                    out_hbm.at[cid, pl.ds(sid * RPS, RPS)])


def _tc_a_body(x_ref, w1_ref, degp_ref, h_ref, dinv_ref):
    deg = (degp_ref[0] + degp_ref[1])[:N] + 1.0
    dinv = lax.rsqrt(deg)
    dinv_ref[...] = dinv
    h = jnp.dot(x_ref[...], w1_ref[...], preferred_element_type=jnp.float32)
    h_ref[...] = h * dinv[:, :1]


def _tc_b_body(raw_ref, h1_ref, dinv_ref, b1_ref, w2_ref, h2_ref):
    dinv = dinv_ref[:, :1]
    raw = (raw_ref[0] + raw_ref[1])[:N]
    z1 = jnp.maximum((raw + h1_ref[...]) * dinv + b1_ref[...], 0.0)
    h2 = jnp.dot(z1, w2_ref[...], preferred_element_type=jnp.float32)
    h2_ref[...] = h2 * dinv


def _tc_c_body(raw_ref, h2_ref, dinv_ref, b2_ref, batch_ref,
               wf1_ref, bf1_ref, wf2_ref, bf2_ref, out_ref):
    dinv = dinv_ref[:, :1]
    raw = (raw_ref[0] + raw_ref[1])[:N]
    z2 = jnp.maximum((raw + h2_ref[...]) * dinv + b2_ref[...], 0.0)
    gids = lax.broadcasted_iota(jnp.int32, (G, N), 0)
    onehot = (gids == batch_ref[...]).astype(jnp.float32)
    sums = jnp.dot(onehot, z2, preferred_element_type=jnp.float32)
    cnts = jnp.sum(onehot, axis=1, keepdims=True)
    pooled = sums / jnp.maximum(cnts, 1.0)
    hfc = jnp.maximum(
        jnp.dot(pooled, wf1_ref[...], preferred_element_type=jnp.float32)
        + bf1_ref[...], 0.0)
    out_ref[...] = (jnp.dot(hfc, wf2_ref[...],
                            preferred_element_type=jnp.float32)
                    + bf2_ref[...])


_tc_a = pl.pallas_call(
    _tc_a_body,
    out_shape=(jax.ShapeDtypeStruct((N, H), jnp.float32),
               jax.ShapeDtypeStruct((N, DEGW), jnp.float32)),
)

_tc_b = pl.pallas_call(
    _tc_b_body,
    out_shape=jax.ShapeDtypeStruct((N, H), jnp.float32),
)

_tc_c = pl.pallas_call(
    _tc_c_body,
    out_shape=jax.ShapeDtypeStruct((G, OUT), jnp.float32),
)


def kernel(x, edge_index, batch, W1, b1, W2, b2, Wf1, bf1, Wf2, bf2):
    src = edge_index[0].reshape(NW, CHW, C)
    dst = edge_index[1].reshape(NW, CHW, C)
    zeros_h = jnp.zeros((NP, H), jnp.float32)
    zeros_d = jnp.zeros((NP, DEGW), jnp.float32)
    ones_d = jnp.ones((C, DEGW), jnp.float32)

    degp = _sc_degree(dst, ones_d, zeros_d)
    h1s, dinv = _tc_a(x, W1, degp)
    raw1 = _sc_scatter(h1s, src, dst, zeros_h)
    h2s = _tc_b(raw1, h1s, dinv, b1.reshape(1, H), W2)
    raw2 = _sc_scatter(h2s, src, dst, zeros_h)
    out = _tc_c(raw2, h2s, dinv, b2.reshape(1, H), batch.reshape(1, N),
                Wf1, bf1.reshape(1, H // 2), Wf2, bf2.reshape(1, OUT))
    return out


# speedup vs baseline: 54.2445x; 1.2495x over previous
import functools

import jax
import jax.numpy as jnp
from jax import lax
from jax.experimental import pallas as pl
from jax.experimental.pallas import tpu as pltpu
from jax.experimental.pallas import tpu_sc as plsc

N = 10000
E = 320000
D = 128
H = 64
G = 64
OUT = 2

C = 125
NW = 32
CHW = E // (NW * C)
NP = 10240
RPS = NP // 16
DEGW = 16
NSLOT = 8

_mesh = plsc.VectorSubcoreMesh(core_axis_name="c", subcore_axis_name="s")
_sc_params = pltpu.CompilerParams(use_tc_tiling_on_sc=False)


@functools.partial(
    pl.kernel,
    out_type=jax.ShapeDtypeStruct((2, NP, DEGW), jnp.float32),
    mesh=_mesh,
    scratch_types=[
        pltpu.VMEM((CHW, C), jnp.int32),
        pltpu.VMEM((C, DEGW), jnp.float32),
        pltpu.VMEM_SHARED((NP, DEGW), jnp.float32),
        [pltpu.SemaphoreType.DMA] * 4,
    ],
    compiler_params=_sc_params,
)
def _sc_degree(dst_hbm, ones_hbm, zeros_hbm, out_hbm, dst_v, ones_v, acc, sems):
    cid = lax.axis_index("c")
    sid = lax.axis_index("s")
    wid = sid * 2 + cid
    pltpu.sync_copy(zeros_hbm.at[pl.ds(sid * RPS, RPS)],
                    acc.at[pl.ds(sid * RPS, RPS)])
    pltpu.sync_copy(dst_hbm.at[wid], dst_v)
    pltpu.sync_copy(ones_hbm, ones_v)
    plsc.subcore_barrier()

    def _scat(j, b):
        pltpu.async_copy(ones_v, acc.at[dst_v.at[j]], sems[b], add=True)

    def _drain(j, b):
        pltpu.make_async_copy(ones_v, acc.at[dst_v.at[j]], sems[b]).wait()

    for b in range(4):
        _scat(b, b)

    def body(g, carry):
        for b in range(4):
            j = 4 + g * 4 + b
            _drain(j, b)
            _scat(j, b)
        return carry

    lax.fori_loop(0, (CHW - 4) // 4, body, 0)
    for b in range(4):
        _drain(0, b)
    plsc.subcore_barrier()
    pltpu.sync_copy(acc.at[pl.ds(sid * RPS, RPS)],
                    out_hbm.at[cid, pl.ds(sid * RPS, RPS)])


@functools.partial(
    pl.kernel,
    out_type=jax.ShapeDtypeStruct((2, NP, H), jnp.bfloat16),
    mesh=_mesh,
    scratch_types=[
        pltpu.VMEM((CHW, C), jnp.int32),
        pltpu.VMEM((CHW, C), jnp.int32),
        pltpu.VMEM((NSLOT, C, H), jnp.bfloat16),
        pltpu.VMEM_SHARED((NP, H), jnp.bfloat16),
        [pltpu.SemaphoreType.DMA] * NSLOT,
        [pltpu.SemaphoreType.DMA] * NSLOT,
    ],
    compiler_params=_sc_params,
)
def _sc_scatter(h_hbm, src_hbm, dst_hbm, zeros_hbm, out_hbm,
                src_v, dst_v, rows_v, acc, gsems, ssems):
    cid = lax.axis_index("c")
    sid = lax.axis_index("s")
    wid = sid * 2 + cid
    pltpu.sync_copy(zeros_hbm.at[pl.ds(sid * RPS, RPS)],
                    acc.at[pl.ds(sid * RPS, RPS)])
    pltpu.sync_copy(src_hbm.at[wid], src_v)
    pltpu.sync_copy(dst_hbm.at[wid], dst_v)
    plsc.subcore_barrier()

    def _gather(j, b):
        pltpu.async_copy(h_hbm.at[src_v.at[j]], rows_v.at[b], gsems[b])

    def _gwait(j, b):
        pltpu.make_async_copy(h_hbm.at[src_v.at[j]], rows_v.at[b],
                              gsems[b]).wait()

    def _scat(j, b):
        pltpu.async_copy(rows_v.at[b], acc.at[dst_v.at[j]], ssems[b], add=True)

    def _swait(j, b):
        pltpu.make_async_copy(rows_v.at[b], acc.at[dst_v.at[j]],
                              ssems[b]).wait()

    for b in range(4):
        _gather(b, b)
    for j in range(4):
        _gwait(j, j)
        _scat(j, j)
        _gather(j + 4, j + 4)

    def body(g, carry):
        for boff in range(8):
            j = 4 + g * 8 + boff
            b = (4 + boff) % NSLOT
            b4 = boff % NSLOT
            _gwait(j, b)
            _scat(j, b)
            _swait(j - 4, b4)
            _gather(j + 4, b4)
        return carry

    lax.fori_loop(0, (CHW - 8) // 8, body, 0)
    for j in range(CHW - 4, CHW):
        b = j % NSLOT
        _gwait(j, b)
        _scat(j, b)
    for b in range(NSLOT):
        _swait(0, b)
    plsc.subcore_barrier()
    pltpu.sync_copy(acc.at[pl.ds(sid * RPS, RPS)],
                    out_hbm.at[cid, pl.ds(sid * RPS, RPS)])


def _tc_a_body(x_ref, w1_ref, degp_ref, h_ref, dinv_ref):
    deg = (degp_ref[0] + degp_ref[1])[:N] + 1.0
    dinv = lax.rsqrt(deg)
    dinv_ref[...] = dinv
    h = jnp.dot(x_ref[...], w1_ref[...], preferred_element_type=jnp.float32)
    h_ref[...] = (h * dinv[:, :1]).astype(jnp.bfloat16)


def _tc_b_body(raw_ref, h1_ref, dinv_ref, b1_ref, w2_ref, h2_ref):
    dinv = dinv_ref[:, :1]
    raw = (raw_ref[0].astype(jnp.float32) + raw_ref[1].astype(jnp.float32))[:N]
    z1 = jnp.maximum((raw + h1_ref[...].astype(jnp.float32)) * dinv
                     + b1_ref[...], 0.0)
    h2 = jnp.dot(z1, w2_ref[...], preferred_element_type=jnp.float32)
    h2_ref[...] = (h2 * dinv).astype(jnp.bfloat16)


def _tc_c_body(raw_ref, h2_ref, dinv_ref, b2_ref, batch_ref,
               wf1_ref, bf1_ref, wf2_ref, bf2_ref, out_ref):
    dinv = dinv_ref[:, :1]
    raw = (raw_ref[0].astype(jnp.float32) + raw_ref[1].astype(jnp.float32))[:N]
    z2 = jnp.maximum((raw + h2_ref[...].astype(jnp.float32)) * dinv
                     + b2_ref[...], 0.0)
    gids = lax.broadcasted_iota(jnp.int32, (G, N), 0)
    onehot = (gids == batch_ref[...]).astype(jnp.float32)
    sums = jnp.dot(onehot, z2, preferred_element_type=jnp.float32)
    cnts = jnp.sum(onehot, axis=1, keepdims=True)
    pooled = sums / jnp.maximum(cnts, 1.0)
    hfc = jnp.maximum(
        jnp.dot(pooled, wf1_ref[...], preferred_element_type=jnp.float32)
        + bf1_ref[...], 0.0)
    out_ref[...] = (jnp.dot(hfc, wf2_ref[...],
                            preferred_element_type=jnp.float32)
                    + bf2_ref[...])


_tc_a = pl.pallas_call(
    _tc_a_body,
    out_shape=(jax.ShapeDtypeStruct((N, H), jnp.bfloat16),
               jax.ShapeDtypeStruct((N, DEGW), jnp.float32)),
)

_tc_b = pl.pallas_call(
    _tc_b_body,
    out_shape=jax.ShapeDtypeStruct((N, H), jnp.bfloat16),
)

_tc_c = pl.pallas_call(
    _tc_c_body,
    out_shape=jax.ShapeDtypeStruct((G, OUT), jnp.float32),
)


def kernel(x, edge_index, batch, W1, b1, W2, b2, Wf1, bf1, Wf2, bf2):
    src = edge_index[0].reshape(NW, CHW, C)
    dst = edge_index[1].reshape(NW, CHW, C)
    zeros_h = jnp.zeros((NP, H), jnp.bfloat16)
    zeros_d = jnp.zeros((NP, DEGW), jnp.float32)
    ones_d = jnp.ones((C, DEGW), jnp.float32)

    degp = _sc_degree(dst, ones_d, zeros_d)
    h1s, dinv = _tc_a(x, W1, degp)
    raw1 = _sc_scatter(h1s, src, dst, zeros_h)
    h2s = _tc_b(raw1, h1s, dinv, b1.reshape(1, H), W2)
    raw2 = _sc_scatter(h2s, src, dst, zeros_h)
    out = _tc_c(raw2, h2s, dinv, b2.reshape(1, H), batch.reshape(1, N),
                Wf1, bf1.reshape(1, H // 2), Wf2, bf2.reshape(1, OUT))
    return out
